# baseline (speedup 1.0000x reference)
import jax
import jax.numpy as jnp
from jax import lax
from jax.experimental import pallas as pl
from jax.experimental.pallas import tpu as pltpu

_F32 = jnp.float32


def _rowsum_t(m):
    tt, c = m.shape
    z = m[:, 0:128]
    for k in range(1, c // 128):
        z = z + m[:, 128 * k:128 * (k + 1)]
    zt = jnp.transpose(z, (1, 0))
    return jnp.sum(zt, axis=0, keepdims=True)


def _pair_sims_row(x):
    tt, c = x.shape
    xs = pltpu.roll(x, tt - 1, 0)
    dotr = _rowsum_t(x * xs)
    n2r = _rowsum_t(x * x)
    nrm = jnp.maximum(jnp.sqrt(n2r), 1e-12)
    denom = nrm * pltpu.roll(nrm, tt - 1, 1)
    return dotr / denom


def _body(m0_ref, m1_ref, w1_ref, b1_ref, w2_ref, b2_ref,
          lg0_ref, lg1_ref, mk0_ref, mk1_ref, theta_ref, acc_ref,
          *, T, tau, theta_min, theta_max, B):
    b = pl.program_id(0)
    x0 = m0_ref[0]
    x1 = m1_ref[0]
    tt = x0.shape[0]

    cs = (jnp.sum(x0, axis=0, keepdims=True)
          + jnp.sum(x1, axis=0, keepdims=True))
    g = cs * (1.0 / T)
    h = jnp.dot(g, w1_ref[...], preferred_element_type=_F32) + b1_ref[...]
    h = 0.5 * h * (1.0 + lax.erf(h * _F32(0.7071067811865476)))
    t2 = jnp.dot(h, w2_ref[...], preferred_element_type=_F32) + b2_ref[...]
    theta = theta_min + (theta_max - theta_min) * jax.nn.sigmoid(t2)
    theta_ref[0] = theta

    inv_tau = _F32(1.0 / max(tau, 1e-6))
    lg0 = (_pair_sims_row(x0) - theta) * inv_tau
    lg1 = (_pair_sims_row(x1) - theta) * inv_tau
    lg0_ref[0] = lg0
    lg1_ref[0] = lg1
    mk0 = (lg0 >= 0).astype(_F32)
    mk1 = (lg1 >= 0).astype(_F32)
    mk0_ref[0] = mk0
    mk1_ref[0] = mk1

    even = (lax.broadcasted_iota(jnp.int32, (1, tt), 1) % 2) == 0
    zero = jnp.zeros((1, tt), _F32)
    nmask = jnp.sum(jnp.where(even, mk0 + mk1, zero), axis=(0, 1), keepdims=True)
    nsig = jnp.sum(jnp.where(even, jax.nn.sigmoid(lg0) + jax.nn.sigmoid(lg1),
                             zero), axis=(0, 1), keepdims=True)
    part = jnp.concatenate([nmask, nsig], axis=1)

    @pl.when(b == 0)
    def _init():
        acc_ref[...] = part

    @pl.when(b != 0)
    def _acc():
        acc_ref[...] += part


def kernel(metric, W1, b1, W2, b2):
    tau_gate = 0.1
    theta_min = 0.0
    theta_max = 2.0
    B, T, C = metric.shape
    if T % 2 == 1:
        metric = metric[:, :-1, :]
        T = T - 1
    P = T // 2
    H = W1.shape[1]
    TT = T // 2

    import functools
    lg0, lg1, mk0, mk1, theta3, acc = pl.pallas_call(
        functools.partial(_body, T=T, tau=tau_gate, theta_min=theta_min,
                          theta_max=theta_max, B=B),
        grid=(B,),
        in_specs=[
            pl.BlockSpec((1, TT, C), lambda b: (b, 0, 0)),
            pl.BlockSpec((1, TT, C), lambda b: (b, 1, 0)),
            pl.BlockSpec((C, H), lambda b: (0, 0)),
            pl.BlockSpec((1, H), lambda b: (0, 0)),
            pl.BlockSpec((H, 1), lambda b: (0, 0)),
            pl.BlockSpec((1, 1), lambda b: (0, 0)),
        ],
        out_specs=[
            pl.BlockSpec((1, 1, TT), lambda b: (b, 0, 0)),
            pl.BlockSpec((1, 1, TT), lambda b: (b, 0, 0)),
            pl.BlockSpec((1, 1, TT), lambda b: (b, 0, 0)),
            pl.BlockSpec((1, 1, TT), lambda b: (b, 0, 0)),
            pl.BlockSpec((1, 1, 1), lambda b: (b, 0, 0)),
            pl.BlockSpec((1, 2), lambda b: (0, 0)),
        ],
        out_shape=[
            jax.ShapeDtypeStruct((B, 1, TT), _F32),
            jax.ShapeDtypeStruct((B, 1, TT), _F32),
            jax.ShapeDtypeStruct((B, 1, TT), _F32),
            jax.ShapeDtypeStruct((B, 1, TT), _F32),
            jax.ShapeDtypeStruct((B, 1, 1), _F32),
            jax.ShapeDtypeStruct((1, 2), _F32),
        ],
    )(metric, metric, W1, b1.reshape(1, H), W2, b2.reshape(1, 1))

    logits = jnp.concatenate(
        [lg0.reshape(B, TT), lg1.reshape(B, TT)], axis=1)[:, ::2]
    maskf = jnp.concatenate(
        [mk0.reshape(B, TT), mk1.reshape(B, TT)], axis=1)[:, ::2]
    n = B * P
    ratio = (acc[0, 0] / n).reshape(())
    mpm = (acc[0, 1] / n).reshape(())
    kre = (1.0 - 0.5 * ratio).reshape(())
    return (logits,
            maskf.astype(bool),
            theta3.reshape(B),
            ratio,
            mpm,
            kre)


# speedup vs baseline: 1.0079x; 1.0079x over previous
import functools

import jax
import jax.numpy as jnp
from jax import lax
from jax.experimental import pallas as pl
from jax.experimental.pallas import tpu as pltpu

_F32 = jnp.float32


def _rowsum_t(m):
    tt, c = m.shape
    z = m[:, 0:128]
    for k in range(1, c // 128):
        z = z + m[:, 128 * k:128 * (k + 1)]
    zt = jnp.transpose(z, (1, 0))
    return jnp.sum(zt, axis=0, keepdims=True)


def _pair_sims_row(x):
    tt, c = x.shape
    xs = pltpu.roll(x, tt - 1, 0)
    dotr = _rowsum_t(x * xs)
    n2r = _rowsum_t(x * x)
    nrm = jnp.maximum(jnp.sqrt(n2r), 1e-12)
    denom = nrm * pltpu.roll(nrm, tt - 1, 1)
    return dotr / denom


def _stage2_body(cs_ref, sim_ref, w1_ref, b1_ref, w2_ref, b2_ref,
                 logits_ref, mask_ref, theta_ref, ratio_ref, mpm_ref, kre_ref,
                 *, T, tau, theta_min, theta_max):
    g = cs_ref[...] * (1.0 / T)
    h = jnp.dot(g, w1_ref[...], preferred_element_type=_F32) + b1_ref[...]
    h = 0.5 * h * (1.0 + lax.erf(h * _F32(0.7071067811865476)))
    t2 = jnp.dot(h, w2_ref[...], preferred_element_type=_F32) + b2_ref[...]
    theta = theta_min + (theta_max - theta_min) * jax.nn.sigmoid(t2)
    theta_ref[...] = theta
    logits = (sim_ref[...] - theta) / max(tau, 1e-6)
    logits_ref[...] = logits
    maskf = (logits >= 0).astype(_F32)
    mask_ref[...] = maskf
    n = logits.shape[0] * logits.shape[1]
    ratio = jnp.sum(maskf, axis=(0, 1), keepdims=True) * (1.0 / n)
    ratio_ref[...] = ratio
    mpm_ref[...] = jnp.sum(jax.nn.sigmoid(logits), axis=(0, 1), keepdims=True) * (1.0 / n)
    kre_ref[...] = 1.0 - 0.5 * ratio


def kernel(metric, W1, b1, W2, b2):
    tau_gate = 0.1
    theta_min = 0.0
    theta_max = 2.0
    B, T, C = metric.shape
    if T % 2 == 1:
        metric = metric[:, :-1, :]
        T = T - 1
    P = T // 2
    H = W1.shape[1]

    TT = T // 4
    NT = T // (2 * TT)

    def _cs_body(m0_ref, m1_ref, sim0_ref, sim1_ref, cs_ref):
        t = pl.program_id(1)
        x0 = m0_ref[0]
        x1 = m1_ref[0]
        sim0_ref[0, 0] = _pair_sims_row(x0)
        sim1_ref[0, 0] = _pair_sims_row(x1)
        g = (jnp.sum(x0, axis=0, keepdims=True)
             + jnp.sum(x1, axis=0, keepdims=True))

        @pl.when(t == 0)
        def _init():
            cs_ref[0] = g

        @pl.when(t != 0)
        def _acc():
            cs_ref[0] += g

    sim0, sim1, colsum = pl.pallas_call(
        _cs_body,
        grid=(B, NT),
        in_specs=[
            pl.BlockSpec((1, TT, C), lambda b, t: (b, 2 * t, 0)),
            pl.BlockSpec((1, TT, C), lambda b, t: (b, 2 * t + 1, 0)),
        ],
        out_specs=[
            pl.BlockSpec((1, 1, 1, TT), lambda b, t: (b, t, 0, 0)),
            pl.BlockSpec((1, 1, 1, TT), lambda b, t: (b, t, 0, 0)),
            pl.BlockSpec((1, 1, C), lambda b, t: (b, 0, 0)),
        ],
        out_shape=[
            jax.ShapeDtypeStruct((B, NT, 1, TT), _F32),
            jax.ShapeDtypeStruct((B, NT, 1, TT), _F32),
            jax.ShapeDtypeStruct((B, 1, C), _F32),
        ],
    )(metric, metric)

    simfull = jnp.stack(
        [sim0.reshape(B, NT, TT), sim1.reshape(B, NT, TT)],
        axis=2).reshape(B, T)
    sim = simfull[:, ::2]
    colsum = colsum.reshape(B, C)

    outs = pl.pallas_call(
        functools.partial(_stage2_body, T=T, tau=tau_gate,
                          theta_min=theta_min, theta_max=theta_max),
        out_shape=[
            jax.ShapeDtypeStruct((B, P), _F32),
            jax.ShapeDtypeStruct((B, P), _F32),
            jax.ShapeDtypeStruct((B, 1), _F32),
            jax.ShapeDtypeStruct((1, 1), _F32),
            jax.ShapeDtypeStruct((1, 1), _F32),
            jax.ShapeDtypeStruct((1, 1), _F32),
        ],
    )(colsum, sim, W1, b1.reshape(1, H), W2, b2.reshape(1, 1))

    logits, maskf, theta2, ratio, mpm, kre = outs
    return (logits,
            maskf.astype(bool),
            theta2.reshape(B),
            ratio.reshape(()),
            mpm.reshape(()),
            kre.reshape(()))


# speedup vs baseline: 1.0968x; 1.0881x over previous
import functools

import jax
import jax.numpy as jnp
from jax import lax
from jax.experimental import pallas as pl
from jax.experimental.pallas import tpu as pltpu

_F32 = jnp.float32


def _rowsum_t(m):
    tt, c = m.shape
    z = m[:, 0:128]
    for k in range(1, c // 128):
        z = z + m[:, 128 * k:128 * (k + 1)]
    zt = jnp.transpose(z, (1, 0))
    return jnp.sum(zt, axis=0, keepdims=True)


def _pair_sims_row(x):
    tt, c = x.shape
    xs = pltpu.roll(x, tt - 1, 0)
    dotr = _rowsum_t(x * xs)
    n2r = _rowsum_t(x * x)
    nrm = jnp.maximum(jnp.sqrt(n2r), 1e-12)
    denom = nrm * pltpu.roll(nrm, tt - 1, 1)
    return dotr / denom


def _stage1_body(m0_ref, m1_ref, sim0_ref, sim1_ref, cs_ref):
    x0 = m0_ref[0]
    x1 = m1_ref[0]
    sim0_ref[0] = _pair_sims_row(x0)
    sim1_ref[0] = _pair_sims_row(x1)
    cs_ref[0] = (jnp.sum(x0, axis=0, keepdims=True)
                 + jnp.sum(x1, axis=0, keepdims=True))


def _stage2_body(cs_ref, sim_ref, w1_ref, b1_ref, w2_ref, b2_ref,
                 logits_ref, mask_ref, theta_ref, ratio_ref, mpm_ref, kre_ref,
                 *, T, tau, theta_min, theta_max):
    g = cs_ref[...] * (1.0 / T)
    h = jnp.dot(g, w1_ref[...], preferred_element_type=_F32) + b1_ref[...]
    h = 0.5 * h * (1.0 + lax.erf(h * _F32(0.7071067811865476)))
    t2 = jnp.dot(h, w2_ref[...], preferred_element_type=_F32) + b2_ref[...]
    theta = theta_min + (theta_max - theta_min) * jax.nn.sigmoid(t2)
    theta_ref[...] = theta
    logits = (sim_ref[...] - theta) / max(tau, 1e-6)
    logits_ref[...] = logits
    maskf = (logits >= 0).astype(_F32)
    mask_ref[...] = maskf
    n = logits.shape[0] * logits.shape[1]
    ratio = jnp.sum(maskf, axis=(0, 1), keepdims=True) * (1.0 / n)
    ratio_ref[...] = ratio
    mpm_ref[...] = jnp.sum(jax.nn.sigmoid(logits), axis=(0, 1), keepdims=True) * (1.0 / n)
    kre_ref[...] = 1.0 - 0.5 * ratio


def kernel(metric, W1, b1, W2, b2):
    tau_gate = 0.1
    theta_min = 0.0
    theta_max = 2.0
    B, T, C = metric.shape
    if T % 2 == 1:
        metric = metric[:, :-1, :]
        T = T - 1
    P = T // 2
    H = W1.shape[1]

    TT = T // 2

    sim0, sim1, colsum = pl.pallas_call(
        _stage1_body,
        grid=(B,),
        in_specs=[
            pl.BlockSpec((1, TT, C), lambda b: (b, 0, 0)),
            pl.BlockSpec((1, TT, C), lambda b: (b, 1, 0)),
        ],
        out_specs=[
            pl.BlockSpec((1, 1, TT), lambda b: (b, 0, 0)),
            pl.BlockSpec((1, 1, TT), lambda b: (b, 0, 0)),
            pl.BlockSpec((1, 1, C), lambda b: (b, 0, 0)),
        ],
        out_shape=[
            jax.ShapeDtypeStruct((B, 1, TT), _F32),
            jax.ShapeDtypeStruct((B, 1, TT), _F32),
            jax.ShapeDtypeStruct((B, 1, C), _F32),
        ],
    )(metric, metric)

    simfull = jnp.concatenate(
        [sim0.reshape(B, TT), sim1.reshape(B, TT)], axis=1)
    sim = simfull[:, ::2]
    colsum = colsum.reshape(B, C)

    outs = pl.pallas_call(
        functools.partial(_stage2_body, T=T, tau=tau_gate,
                          theta_min=theta_min, theta_max=theta_max),
        out_shape=[
            jax.ShapeDtypeStruct((B, P), _F32),
            jax.ShapeDtypeStruct((B, P), _F32),
            jax.ShapeDtypeStruct((B, 1), _F32),
            jax.ShapeDtypeStruct((1, 1), _F32),
            jax.ShapeDtypeStruct((1, 1), _F32),
            jax.ShapeDtypeStruct((1, 1), _F32),
        ],
    )(colsum, sim, W1, b1.reshape(1, H), W2, b2.reshape(1, 1))

    logits, maskf, theta2, ratio, mpm, kre = outs
    return (logits,
            maskf.astype(bool),
            theta2.reshape(B),
            ratio.reshape(()),
            mpm.reshape(()),
            kre.reshape(()))
